# Initial kernel scaffold; baseline (speedup 1.0000x reference)
import jax
import jax.numpy as jnp
from jax.experimental import pallas as pl


def kernel(h, edge_index, tfidfembed, W_fc, W_feat, W_attn):
    raise NotImplementedError("write your pallas kernel here")


# speedup vs baseline: 8.8231x; 8.8231x over previous
import functools

import jax
import jax.numpy as jnp
from jax import lax
from jax.experimental import pallas as pl
from jax.experimental.pallas import tpu as pltpu
from jax.experimental.pallas import tpu_sc as plsc

N_W = 5000
N_S = 5000
E = 320000
OUT = 128
FEAT = 16

NP = 5120
NCORES = 2
NSUB = 16
NWORK = NCORES * NSUB
EP = 327680
EW = EP // NWORK
CB = 128
NB = EW // CB
DEN_ROWS = 48

NEG_BIG = -1e30


def _dense_body(h_ref, wfcT_ref, tf_ref, wattn_ref, wfeat_ref,
                z_ref, s_ref, f_ref):
    a1 = wattn_ref[0, :OUT]
    a3 = wattn_ref[0, 2 * OUT:3 * OUT]
    z = jnp.dot(h_ref[...], wfcT_ref[...], preferred_element_type=jnp.float32)
    z_ref[...] = z
    s_ref[...] = jnp.sum(z * a1[None, :], axis=1)
    w3 = jnp.sum(wfeat_ref[...] * a3[:, None], axis=0)
    fb = jnp.sum(tf_ref[...] * w3[None, :], axis=1)
    nrows = fb.shape[0]
    rows = pl.program_id(0) * nrows + lax.iota(jnp.int32, nrows)
    f_ref[...] = jnp.where(rows < E, fb, NEG_BIG)


def _dense_call(h_p, wfcT, tfidf_p, wattn, wfeat):
    grid = 20
    zb = NP // grid
    fb = EP // grid
    return pl.pallas_call(
        _dense_body,
        grid=(grid,),
        in_specs=[
            pl.BlockSpec((zb, OUT), lambda i: (i, 0)),
            pl.BlockSpec((OUT, OUT), lambda i: (0, 0)),
            pl.BlockSpec((fb, FEAT), lambda i: (i, 0)),
            pl.BlockSpec((1, 3 * OUT), lambda i: (0, 0)),
            pl.BlockSpec((OUT, FEAT), lambda i: (0, 0)),
        ],
        out_specs=[
            pl.BlockSpec((zb, OUT), lambda i: (i, 0)),
            pl.BlockSpec((zb,), lambda i: (i,)),
            pl.BlockSpec((fb,), lambda i: (i,)),
        ],
        out_shape=[
            jax.ShapeDtypeStruct((NP, OUT), jnp.float32),
            jax.ShapeDtypeStruct((NP,), jnp.float32),
            jax.ShapeDtypeStruct((EP,), jnp.float32),
        ],
    )(h_p, wfcT, tfidf_p, wattn, wfeat)


def _edge_body(src_hbm, dst_hbm, f_hbm, ssrc_hbm, z_hbm,
               out_hbm, den_hbm,
               s_tab, den_tab, src_v, dst_v, f_v, ex_v, rows_v,
               sh_out, sh_den, iota_v, sem):
    cid = lax.axis_index("c")
    sid = lax.axis_index("s")
    wid = sid * NCORES + cid

    pltpu.sync_copy(ssrc_hbm, s_tab)

    zero16 = jnp.zeros((16,), jnp.float32)

    def _zero_den(r, _):
        for j in range(8):
            den_tab[r, pl.ds(j * 16, 16)] = zero16
        return 0
    lax.fori_loop(0, DEN_ROWS, _zero_den, 0)

    def _zero_rows(i, _):
        for j in range(8):
            rows_v[i, pl.ds(j * 16, 16)] = zero16
        return 0
    lax.fori_loop(0, CB, _zero_rows, 0)

    r0 = sid * (NP // NSUB)
    pltpu.sync_copy(rows_v, sh_out.at[pl.ds(r0, 128)])
    pltpu.sync_copy(rows_v, sh_out.at[pl.ds(r0 + 128, 128)])
    pltpu.sync_copy(rows_v.at[pl.ds(0, 64)], sh_out.at[pl.ds(r0 + 256, 64)])

    @pl.when(sid == 0)
    def _():
        pltpu.sync_copy(den_tab, sh_den)

    ii = lax.iota(jnp.int32, 16)
    iota_v[0, pl.ds(0, 16)] = ii
    iota_v[0, pl.ds(16, 16)] = ii + 16
    iota_v[0, pl.ds(32, 16)] = ii + 32

    plsc.subcore_barrier()

    def _block(b, _):
        base = wid * EW + b * CB
        pltpu.sync_copy(src_hbm.at[pl.ds(base, CB)], src_v)
        pltpu.sync_copy(dst_hbm.at[pl.ds(base, CB)], dst_v.at[0])
        pltpu.sync_copy(f_hbm.at[pl.ds(base, CB)], f_v)
        gather = pltpu.async_copy(z_hbm.at[src_v], rows_v, sem)

        for g in range(CB // 16):
            sl = pl.ds(g * 16, 16)
            idx16 = src_v[sl]
            s16 = plsc.load_gather(s_tab, [idx16])
            x = s16 + f_v[sl]
            e = jnp.maximum(x, x * 0.01)
            ex = jnp.exp(e)
            ex_v[sl] = ex
            d16 = dst_v[0, sl]
            plsc.addupdate_scatter(
                den_tab,
                [lax.shift_right_logical(d16, 7), lax.bitwise_and(d16, 127)],
                ex)

        gather.wait()

        def _scale(g, _):
            ex16 = ex_v[pl.ds(g * 16, 16)]
            for l in range(16):
                i = g * 16 + l
                vx = jnp.full((16,), ex16[l], jnp.float32)
                for j in range(8):
                    sl2 = pl.ds(j * 16, 16)
                    rows_v[i, sl2] = rows_v[i, sl2] * vx
            return 0
        lax.fori_loop(0, CB // 16, _scale, 0)

        pltpu.sync_copy(rows_v, sh_out.at[dst_v.at[0]], add=True)
        return 0

    lax.fori_loop(0, NB, _block, 0)

    plsc.subcore_barrier()

    pltpu.sync_copy(den_tab, sh_den.at[iota_v.at[0]], add=True)

    plsc.subcore_barrier()

    pltpu.sync_copy(sh_out.at[pl.ds(r0, NP // NSUB)],
                    out_hbm.at[cid, pl.ds(r0, NP // NSUB)])

    @pl.when(sid == 0)
    def _():
        pltpu.sync_copy(sh_den, den_hbm.at[cid])


def _edge_call(src_p, dst_p, f_p, ssrc, z):
    mesh = plsc.VectorSubcoreMesh(core_axis_name="c", subcore_axis_name="s")
    fn = pl.kernel(
        _edge_body,
        out_type=[
            jax.ShapeDtypeStruct((NCORES, NP, OUT), jnp.float32),
            jax.ShapeDtypeStruct((NCORES, DEN_ROWS, 128), jnp.float32),
        ],
        mesh=mesh,
        scratch_types=[
            pltpu.VMEM((NP,), jnp.float32),
            pltpu.VMEM((DEN_ROWS, 128), jnp.float32),
            pltpu.VMEM((CB,), jnp.int32),
            pltpu.VMEM((1, CB), jnp.int32),
            pltpu.VMEM((CB,), jnp.float32),
            pltpu.VMEM((CB,), jnp.float32),
            pltpu.VMEM((CB, OUT), jnp.float32),
            pltpu.VMEM_SHARED((NP, OUT), jnp.float32),
            pltpu.VMEM_SHARED((DEN_ROWS, 128), jnp.float32),
            pltpu.VMEM((1, DEN_ROWS), jnp.int32),
            pltpu.SemaphoreType.DMA,
        ],
        compiler_params=pltpu.CompilerParams(needs_layout_passes=False),
    )
    return fn(src_p, dst_p, f_p, ssrc, z)


def _norm_body(p_ref, d_ref, o_ref):
    p = p_ref[0] + p_ref[1]
    d = d_ref[0] + d_ref[1]
    d = jnp.where(d > 0.0, d, 1.0)
    o_ref[...] = p * (1.0 / d)[:, None]


def _norm_call(outp, den):
    grid = 20
    rb = NP // grid
    return pl.pallas_call(
        _norm_body,
        grid=(grid,),
        in_specs=[
            pl.BlockSpec((NCORES, rb, OUT), lambda i: (0, i, 0)),
            pl.BlockSpec((NCORES, rb), lambda i: (0, i)),
        ],
        out_specs=pl.BlockSpec((rb, OUT), lambda i: (i, 0)),
        out_shape=jax.ShapeDtypeStruct((NP, OUT), jnp.float32),
    )(outp, den)


def kernel(h, edge_index, tfidfembed, W_fc, W_feat, W_attn):
    src = edge_index[0]
    dst = edge_index[1]
    h_p = jnp.pad(h[:N_W], ((0, NP - N_W), (0, 0)))
    tf_p = jnp.pad(tfidfembed, ((0, EP - E), (0, 0)))
    z, ssrc, f_p = _dense_call(h_p, W_fc.T, tf_p, W_attn, W_feat)
    src_p = jnp.pad(src, (0, EP - E))
    dst_p = jnp.pad(dst, (0, EP - E))
    outp, denp = _edge_call(src_p, dst_p, f_p, ssrc, z)
    den = denp.reshape(NCORES, DEN_ROWS * 128)[:, :NP]
    res = _norm_call(outp, den)
    return res[:N_S]


# speedup vs baseline: 10.9219x; 1.2379x over previous
import functools

import jax
import jax.numpy as jnp
from jax import lax
from jax.experimental import pallas as pl
from jax.experimental.pallas import tpu as pltpu
from jax.experimental.pallas import tpu_sc as plsc

N_W = 5000
N_S = 5000
E = 320000
OUT = 128
FEAT = 16

NP = 5120
NCORES = 2
NSUB = 16
NWORK = NCORES * NSUB
EP = 327680
EW = EP // NWORK
CB = 64
NB = EW // CB
DEN_ROWS = 48

NEG_BIG = -1e30


def _dense_body(h_ref, wfcT_ref, tf_ref, wattn_ref, wfeat_ref,
                z_ref, s_ref, f_ref):
    a1 = wattn_ref[0, :OUT]
    a3 = wattn_ref[0, 2 * OUT:3 * OUT]
    z = jnp.dot(h_ref[...], wfcT_ref[...], preferred_element_type=jnp.float32)
    z_ref[...] = z
    s_ref[...] = jnp.sum(z * a1[None, :], axis=1)
    w3 = jnp.sum(wfeat_ref[...] * a3[:, None], axis=0)
    fb = jnp.sum(tf_ref[...] * w3[None, :], axis=1)
    nrows = fb.shape[0]
    rows = pl.program_id(0) * nrows + lax.iota(jnp.int32, nrows)
    f_ref[...] = jnp.where(rows < E, fb, NEG_BIG)


def _dense_call(h_p, wfcT, tfidf_p, wattn, wfeat):
    grid = 20
    zb = NP // grid
    fb = EP // grid
    return pl.pallas_call(
        _dense_body,
        grid=(grid,),
        in_specs=[
            pl.BlockSpec((zb, OUT), lambda i: (i, 0)),
            pl.BlockSpec((OUT, OUT), lambda i: (0, 0)),
            pl.BlockSpec((fb, FEAT), lambda i: (i, 0)),
            pl.BlockSpec((1, 3 * OUT), lambda i: (0, 0)),
            pl.BlockSpec((OUT, FEAT), lambda i: (0, 0)),
        ],
        out_specs=[
            pl.BlockSpec((zb, OUT), lambda i: (i, 0)),
            pl.BlockSpec((zb,), lambda i: (i,)),
            pl.BlockSpec((fb,), lambda i: (i,)),
        ],
        out_shape=[
            jax.ShapeDtypeStruct((NP, OUT), jnp.float32),
            jax.ShapeDtypeStruct((NP,), jnp.float32),
            jax.ShapeDtypeStruct((EP,), jnp.float32),
        ],
    )(h_p, wfcT, tfidf_p, wattn, wfeat)


def _edge_body(src_hbm, dst_hbm, f_hbm, ssrc_hbm, z_hbm,
               out_hbm, den_hbm,
               s_tab, den_tab, src_all, dst_all, ex_all,
               g0, g1, sb0, sb1,
               sh_out, sh_den, iota_v,
               gsem0, gsem1, ssem0, ssem1):
    cid = lax.axis_index("c")
    sid = lax.axis_index("s")
    wid = sid * NCORES + cid
    ebase = wid * EW

    pltpu.sync_copy(ssrc_hbm, s_tab)
    pltpu.sync_copy(src_hbm.at[pl.ds(ebase, EW)], src_all)
    pltpu.sync_copy(dst_hbm.at[pl.ds(wid * NB, NB)], dst_all)
    pltpu.sync_copy(f_hbm.at[pl.ds(ebase, EW)], ex_all)

    zero16 = jnp.zeros((16,), jnp.float32)

    def _zero_den(r, _):
        for j in range(8):
            den_tab[r, pl.ds(j * 16, 16)] = zero16
        return 0
    lax.fori_loop(0, DEN_ROWS, _zero_den, 0)

    def _zero_rows(i, _):
        for j in range(8):
            g0[i, pl.ds(j * 16, 16)] = zero16
        return 0
    lax.fori_loop(0, CB, _zero_rows, 0)

    r0 = sid * (NP // NSUB)
    for c in range((NP // NSUB) // CB):
        pltpu.sync_copy(g0, sh_out.at[pl.ds(r0 + c * CB, CB)])

    @pl.when(sid == 0)
    def _():
        pltpu.sync_copy(den_tab, sh_den)

    ii = lax.iota(jnp.int32, 16)
    iota_v[0, pl.ds(0, 16)] = ii
    iota_v[0, pl.ds(16, 16)] = ii + 16
    iota_v[0, pl.ds(32, 16)] = ii + 32

    def _expass(b, _):
        for g in range(CB // 16):
            sl = pl.ds(b * CB + g * 16, 16)
            idx16 = src_all[sl]
            s16 = plsc.load_gather(s_tab, [idx16])
            x = s16 + ex_all[sl]
            e = jnp.maximum(x, x * 0.01)
            ex = jnp.exp(e)
            ex_all[sl] = ex
            d16 = dst_all[b, pl.ds(g * 16, 16)]
            plsc.addupdate_scatter(
                den_tab,
                [lax.shift_right_logical(d16, 7), lax.bitwise_and(d16, 127)],
                ex)
        return 0
    lax.fori_loop(0, NB, _expass, 0)

    plsc.subcore_barrier()

    gbufs = (g0, g1)
    sbufs = (sb0, sb1)
    gsems = (gsem0, gsem1)
    ssems = (ssem0, ssem1)

    def _start_gather(j, b):
        pltpu.async_copy(
            z_hbm.at[src_all.at[pl.ds(b * CB, CB)]], gbufs[j], gsems[j])

    def _scale(j, b):
        gb, sb = gbufs[j], sbufs[j]

        def body(g, _):
            ex16 = ex_all[pl.ds(b * CB + g * 16, 16)]
            for l in range(16):
                i = g * 16 + l
                vx = jnp.full((16,), ex16[l], jnp.float32)
                for jj in range(8):
                    sl2 = pl.ds(jj * 16, 16)
                    sb[i, sl2] = gb[i, sl2] * vx
            return 0
        lax.fori_loop(0, CB // 16, body, 0)

    _start_gather(0, 0)
    _start_gather(1, 1)

    def _pair(k, _):
        for j in range(2):
            b = 2 * k + j
            pltpu.make_async_copy(
                z_hbm.at[src_all.at[pl.ds(b * CB, CB)]],
                gbufs[j], gsems[j]).wait()
            @pl.when(k > 0)
            def _():
                pltpu.make_async_copy(
                    sbufs[j], sh_out.at[dst_all.at[b - 2]], ssems[j]).wait()
            _scale(j, b)
            pltpu.async_copy(
                sbufs[j], sh_out.at[dst_all.at[b]], ssems[j], add=True)
            @pl.when(b + 2 < NB)
            def _():
                pltpu.async_copy(
                    z_hbm.at[src_all.at[pl.ds((b + 2) * CB, CB)]],
                    gbufs[j], gsems[j])
        return 0
    lax.fori_loop(0, NB // 2, _pair, 0)

    for j in range(2):
        pltpu.make_async_copy(
            sbufs[j], sh_out.at[dst_all.at[NB - 2 + j]], ssems[j]).wait()

    plsc.subcore_barrier()

    pltpu.sync_copy(den_tab, sh_den.at[iota_v.at[0]], add=True)

    plsc.subcore_barrier()

    pltpu.sync_copy(sh_out.at[pl.ds(r0, NP // NSUB)],
                    out_hbm.at[cid, pl.ds(r0, NP // NSUB)])

    @pl.when(sid == 0)
    def _():
        pltpu.sync_copy(sh_den, den_hbm.at[cid])


def _edge_call(src_p, dst_p, f_p, ssrc, z):
    mesh = plsc.VectorSubcoreMesh(core_axis_name="c", subcore_axis_name="s")
    fn = pl.kernel(
        _edge_body,
        out_type=[
            jax.ShapeDtypeStruct((NCORES, NP, OUT), jnp.float32),
            jax.ShapeDtypeStruct((NCORES, DEN_ROWS, 128), jnp.float32),
        ],
        mesh=mesh,
        scratch_types=[
            pltpu.VMEM((NP,), jnp.float32),
            pltpu.VMEM((DEN_ROWS, 128), jnp.float32),
            pltpu.VMEM((EW,), jnp.int32),
            pltpu.VMEM((NB, CB), jnp.int32),
            pltpu.VMEM((EW,), jnp.float32),
            pltpu.VMEM((CB, OUT), jnp.float32),
            pltpu.VMEM((CB, OUT), jnp.float32),
            pltpu.VMEM((CB, OUT), jnp.float32),
            pltpu.VMEM((CB, OUT), jnp.float32),
            pltpu.VMEM_SHARED((NP, OUT), jnp.float32),
            pltpu.VMEM_SHARED((DEN_ROWS, 128), jnp.float32),
            pltpu.VMEM((1, DEN_ROWS), jnp.int32),
            pltpu.SemaphoreType.DMA,
            pltpu.SemaphoreType.DMA,
            pltpu.SemaphoreType.DMA,
            pltpu.SemaphoreType.DMA,
        ],
        compiler_params=pltpu.CompilerParams(needs_layout_passes=False),
    )
    return fn(src_p, dst_p, f_p, ssrc, z)


def _norm_body(p_ref, d_ref, o_ref):
    p = p_ref[0] + p_ref[1]
    d = d_ref[0] + d_ref[1]
    d = jnp.where(d > 0.0, d, 1.0)
    o_ref[...] = p * (1.0 / d)[:, None]


def _norm_call(outp, den):
    grid = 20
    rb = NP // grid
    return pl.pallas_call(
        _norm_body,
        grid=(grid,),
        in_specs=[
            pl.BlockSpec((NCORES, rb, OUT), lambda i: (0, i, 0)),
            pl.BlockSpec((NCORES, rb), lambda i: (0, i)),
        ],
        out_specs=pl.BlockSpec((rb, OUT), lambda i: (i, 0)),
        out_shape=jax.ShapeDtypeStruct((NP, OUT), jnp.float32),
    )(outp, den)


def kernel(h, edge_index, tfidfembed, W_fc, W_feat, W_attn):
    src = edge_index[0]
    dst = edge_index[1]
    h_p = jnp.pad(h[:N_W], ((0, NP - N_W), (0, 0)))
    tf_p = jnp.pad(tfidfembed, ((0, EP - E), (0, 0)))
    z, ssrc, f_p = _dense_call(h_p, W_fc.T, tf_p, W_attn, W_feat)
    src_p = jnp.pad(src, (0, EP - E))
    dst_p = jnp.pad(dst, (0, EP - E)).reshape(EP // CB, CB)
    outp, denp = _edge_call(src_p, dst_p, f_p, ssrc, z)
    den = denp.reshape(NCORES, DEN_ROWS * 128)[:, :NP]
    res = _norm_call(outp, den)
    return res[:N_S]


# speedup vs baseline: 12.6843x; 1.1614x over previous
import functools

import jax
import jax.numpy as jnp
from jax import lax
from jax.experimental import pallas as pl
from jax.experimental.pallas import tpu as pltpu
from jax.experimental.pallas import tpu_sc as plsc

N_W = 5000
N_S = 5000
E = 320000
OUT = 128
FEAT = 16

NP = 5120
NCORES = 2
NSUB = 16
NWORK = NCORES * NSUB
EP = 327680
EW = EP // NWORK
CB = 64
NB = EW // CB
DEN_ROWS = 48

NEG_BIG = -1e30


def _dense_body(h_ref, wfcT_ref, tf_ref, wattn_ref, wfeat_ref,
                z_ref, s_ref, f_ref):
    a1 = wattn_ref[0, :OUT]
    a3 = wattn_ref[0, 2 * OUT:3 * OUT]
    z = jnp.dot(h_ref[...], wfcT_ref[...], preferred_element_type=jnp.float32)
    z_ref[...] = z
    s_ref[...] = jnp.sum(z * a1[None, :], axis=1)
    w3 = jnp.sum(wfeat_ref[...] * a3[:, None], axis=0)
    fb = jnp.sum(tf_ref[...] * w3[None, :], axis=1)
    nrows = fb.shape[0]
    rows = pl.program_id(0) * nrows + lax.iota(jnp.int32, nrows)
    f_ref[...] = jnp.where(rows < E, fb, NEG_BIG)


def _dense_call(h_p, wfcT, tfidf_p, wattn, wfeat):
    grid = 20
    zb = NP // grid
    fb = EP // grid
    return pl.pallas_call(
        _dense_body,
        grid=(grid,),
        in_specs=[
            pl.BlockSpec((zb, OUT), lambda i: (i, 0)),
            pl.BlockSpec((OUT, OUT), lambda i: (0, 0)),
            pl.BlockSpec((fb, FEAT), lambda i: (i, 0)),
            pl.BlockSpec((1, 3 * OUT), lambda i: (0, 0)),
            pl.BlockSpec((OUT, FEAT), lambda i: (0, 0)),
        ],
        out_specs=[
            pl.BlockSpec((zb, OUT), lambda i: (i, 0)),
            pl.BlockSpec((zb,), lambda i: (i,)),
            pl.BlockSpec((fb,), lambda i: (i,)),
        ],
        out_shape=[
            jax.ShapeDtypeStruct((NP, OUT), jnp.float32),
            jax.ShapeDtypeStruct((NP,), jnp.float32),
            jax.ShapeDtypeStruct((EP,), jnp.float32),
        ],
    )(h_p, wfcT, tfidf_p, wattn, wfeat)


def _edge_body(src_hbm, dst_hbm, f_hbm, ssrc_hbm, z_hbm,
               out_hbm, den_hbm,
               s_tab, den_tab, src_all, dst_all, ex_all,
               g0, g1, sb0, sb1,
               sh_out, sh_den, iota_v,
               gsem0, gsem1, ssem0, ssem1):
    cid = lax.axis_index("c")
    sid = lax.axis_index("s")
    wid = sid * NCORES + cid
    ebase = wid * EW

    pltpu.sync_copy(ssrc_hbm, s_tab)
    pltpu.sync_copy(src_hbm.at[pl.ds(ebase, EW)], src_all)
    pltpu.sync_copy(dst_hbm.at[pl.ds(wid * NB, NB)], dst_all)
    pltpu.sync_copy(f_hbm.at[pl.ds(ebase, EW)], ex_all)

    zero16 = jnp.zeros((16,), jnp.float32)

    def _zero_den(r, _):
        for j in range(8):
            den_tab[r, pl.ds(j * 16, 16)] = zero16
        return 0
    lax.fori_loop(0, DEN_ROWS, _zero_den, 0)

    def _zero_rows(i, _):
        for j in range(8):
            g0[i, pl.ds(j * 16, 16)] = zero16
        return 0
    lax.fori_loop(0, CB, _zero_rows, 0)

    r0 = sid * (NP // NSUB)
    for c in range((NP // NSUB) // CB):
        pltpu.sync_copy(g0, sh_out.at[pl.ds(r0 + c * CB, CB)])

    @pl.when(sid == 0)
    def _():
        pltpu.sync_copy(den_tab, sh_den)

    ii = lax.iota(jnp.int32, 16)
    iota_v[0, pl.ds(0, 16)] = ii
    iota_v[0, pl.ds(16, 16)] = ii + 16
    iota_v[0, pl.ds(32, 16)] = ii + 32

    def _expass(b, _):
        for g in range(CB // 16):
            sl = pl.ds(b * CB + g * 16, 16)
            idx16 = src_all[sl]
            s16 = plsc.load_gather(s_tab, [idx16])
            x = s16 + ex_all[sl]
            e = jnp.maximum(x, x * 0.01)
            ex = jnp.exp(e)
            ex_all[sl] = ex
            d16 = dst_all[b, pl.ds(g * 16, 16)]
            plsc.addupdate_scatter(
                den_tab,
                [lax.shift_right_logical(d16, 7), lax.bitwise_and(d16, 127)],
                ex)
        return 0
    with jax.named_scope("sc_expass"):
        lax.fori_loop(0, NB, _expass, 0)

    plsc.subcore_barrier()

    gbufs = (g0, g1)
    sbufs = (sb0, sb1)
    gsems = (gsem0, gsem1)
    ssems = (ssem0, ssem1)

    def _start_gather(j, b):
        pltpu.async_copy(
            z_hbm.at[src_all.at[pl.ds(b * CB, CB)]], gbufs[j], gsems[j])

    def _scale(j, b):
        gb, sb = gbufs[j], sbufs[j]

        def body(g, _):
            ex16 = ex_all[pl.ds(b * CB + g * 16, 16)]
            for l in range(16):
                i = g * 16 + l
                vx = jnp.full((16,), ex16[l], jnp.float32)
                for jj in range(8):
                    sl2 = pl.ds(jj * 16, 16)
                    sb[i, sl2] = gb[i, sl2] * vx
            return 0
        lax.fori_loop(0, CB // 16, body, 0)

    _start_gather(0, 0)
    _start_gather(1, 1)

    def _pair(k, _):
        for j in range(2):
            b = 2 * k + j
            pltpu.make_async_copy(
                z_hbm.at[src_all.at[pl.ds(b * CB, CB)]],
                gbufs[j], gsems[j]).wait()
            @pl.when(k > 0)
            def _():
                pltpu.make_async_copy(
                    sbufs[j], sh_out.at[dst_all.at[b - 2]], ssems[j]).wait()
            _scale(j, b)
            pltpu.async_copy(
                sbufs[j], sh_out.at[dst_all.at[b]], ssems[j], add=True)
            @pl.when(b + 2 < NB)
            def _():
                pltpu.async_copy(
                    z_hbm.at[src_all.at[pl.ds((b + 2) * CB, CB)]],
                    gbufs[j], gsems[j])
        return 0
    with jax.named_scope("sc_rowpipe"):
        lax.fori_loop(0, NB // 2, _pair, 0)

        for j in range(2):
            pltpu.make_async_copy(
                sbufs[j], sh_out.at[dst_all.at[NB - 2 + j]], ssems[j]).wait()

    plsc.subcore_barrier()

    pltpu.sync_copy(den_tab, sh_den.at[iota_v.at[0]], add=True)

    plsc.subcore_barrier()

    pltpu.sync_copy(sh_out.at[pl.ds(r0, NP // NSUB)],
                    out_hbm.at[cid, pl.ds(r0, NP // NSUB)])

    @pl.when(sid == 0)
    def _():
        pltpu.sync_copy(sh_den, den_hbm.at[cid])


def _edge_call(src_p, dst_p, f_p, ssrc, z):
    mesh = plsc.VectorSubcoreMesh(core_axis_name="c", subcore_axis_name="s")
    fn = pl.kernel(
        _edge_body,
        out_type=[
            jax.ShapeDtypeStruct((NCORES, NP, OUT), jnp.float32),
            jax.ShapeDtypeStruct((NCORES, DEN_ROWS, 128), jnp.float32),
        ],
        mesh=mesh,
        scratch_types=[
            pltpu.VMEM((NP,), jnp.float32),
            pltpu.VMEM((DEN_ROWS, 128), jnp.float32),
            pltpu.VMEM((EW,), jnp.int32),
            pltpu.VMEM((NB, CB), jnp.int32),
            pltpu.VMEM((EW,), jnp.float32),
            pltpu.VMEM((CB, OUT), jnp.float32),
            pltpu.VMEM((CB, OUT), jnp.float32),
            pltpu.VMEM((CB, OUT), jnp.float32),
            pltpu.VMEM((CB, OUT), jnp.float32),
            pltpu.VMEM_SHARED((NP, OUT), jnp.float32),
            pltpu.VMEM_SHARED((DEN_ROWS, 128), jnp.float32),
            pltpu.VMEM((1, DEN_ROWS), jnp.int32),
            pltpu.SemaphoreType.DMA,
            pltpu.SemaphoreType.DMA,
            pltpu.SemaphoreType.DMA,
            pltpu.SemaphoreType.DMA,
        ],
        compiler_params=pltpu.CompilerParams(
            needs_layout_passes=False, use_tc_tiling_on_sc=True),
    )
    return fn(src_p, dst_p, f_p, ssrc, z)


def _norm_body(p_ref, d_ref, o_ref):
    p = p_ref[0] + p_ref[1]
    d = d_ref[0] + d_ref[1]
    d = jnp.where(d > 0.0, d, 1.0)
    o_ref[...] = p * (1.0 / d)[:, None]


def _norm_call(outp, den):
    grid = 20
    rb = NP // grid
    return pl.pallas_call(
        _norm_body,
        grid=(grid,),
        in_specs=[
            pl.BlockSpec((NCORES, rb, OUT), lambda i: (0, i, 0)),
            pl.BlockSpec((NCORES, rb), lambda i: (0, i)),
        ],
        out_specs=pl.BlockSpec((rb, OUT), lambda i: (i, 0)),
        out_shape=jax.ShapeDtypeStruct((NP, OUT), jnp.float32),
    )(outp, den)


def kernel(h, edge_index, tfidfembed, W_fc, W_feat, W_attn):
    src = edge_index[0]
    dst = edge_index[1]
    h_p = jnp.pad(h[:N_W], ((0, NP - N_W), (0, 0)))
    z, ssrc, f_p = _dense_call(h_p, W_fc.T, tfidfembed, W_attn, W_feat)
    src_p = jnp.pad(src, (0, EP - E))
    dst_p = jnp.pad(dst, (0, EP - E)).reshape(EP // CB, CB)
    outp, denp = _edge_call(src_p, dst_p, f_p, ssrc, z)
    den = denp.reshape(NCORES, DEN_ROWS * 128)[:, :NP]
    res = _norm_call(outp, den)
    return res[:N_S]


# speedup vs baseline: 21.6985x; 1.7107x over previous
import functools

import jax
import jax.numpy as jnp
from jax import lax
from jax.experimental import pallas as pl
from jax.experimental.pallas import tpu as pltpu
from jax.experimental.pallas import tpu_sc as plsc

N_W = 5000
N_S = 5000
E = 320000
OUT = 128
FEAT = 16

NP = 5120
NCORES = 2
NSUB = 16
NWORK = NCORES * NSUB
EP = 327680
EW = EP // NWORK
CB = 64
NB = EW // CB
DEN_ROWS = 48

NEG_BIG = -1e30


def _dense_body(h_ref, wfcT_ref, tf_ref, wattn_ref, wfeat_ref,
                z_ref, s_ref, f_ref):
    a1 = wattn_ref[0, :OUT]
    a3 = wattn_ref[0, 2 * OUT:3 * OUT]
    z = jnp.dot(h_ref[...], wfcT_ref[...], preferred_element_type=jnp.float32)
    z_ref[...] = z
    s_ref[...] = jnp.sum(z * a1[None, :], axis=1)
    w3 = jnp.sum(wfeat_ref[...] * a3[:, None], axis=0)
    fb = jnp.sum(tf_ref[...] * w3[None, :], axis=1)
    nrows = fb.shape[0]
    rows = pl.program_id(0) * nrows + lax.iota(jnp.int32, nrows)
    f_ref[...] = jnp.where(rows < E, fb, NEG_BIG)


def _dense_call(h_p, wfcT, tfidf_p, wattn, wfeat):
    grid = 20
    zb = NP // grid
    fb = EP // grid
    return pl.pallas_call(
        _dense_body,
        grid=(grid,),
        in_specs=[
            pl.BlockSpec((zb, OUT), lambda i: (i, 0)),
            pl.BlockSpec((OUT, OUT), lambda i: (0, 0)),
            pl.BlockSpec((fb, FEAT), lambda i: (i, 0)),
            pl.BlockSpec((1, 3 * OUT), lambda i: (0, 0)),
            pl.BlockSpec((OUT, FEAT), lambda i: (0, 0)),
        ],
        out_specs=[
            pl.BlockSpec((zb, OUT), lambda i: (i, 0)),
            pl.BlockSpec((zb,), lambda i: (i,)),
            pl.BlockSpec((fb,), lambda i: (i,)),
        ],
        out_shape=[
            jax.ShapeDtypeStruct((NP, OUT), jnp.float32),
            jax.ShapeDtypeStruct((NP,), jnp.float32),
            jax.ShapeDtypeStruct((EP,), jnp.float32),
        ],
    )(h_p, wfcT, tfidf_p, wattn, wfeat)


def _edge_body(src_hbm, dst_hbm, f_hbm, ssrc_hbm, z_hbm,
               out_hbm, den_hbm,
               s_tab, den_tab, src_all, dst_all, ex_all,
               g0, g1, sb0, sb1,
               sh_out, sh_den, iota_v,
               gsem0, gsem1, ssem0, ssem1):
    cid = lax.axis_index("c")
    sid = lax.axis_index("s")
    wid = sid * NCORES + cid
    ebase = wid * EW

    pltpu.sync_copy(ssrc_hbm, s_tab)
    pltpu.sync_copy(src_hbm.at[pl.ds(ebase, EW)], src_all)
    pltpu.sync_copy(dst_hbm.at[pl.ds(wid * NB, NB)], dst_all)
    pltpu.sync_copy(f_hbm.at[pl.ds(ebase, EW)], ex_all)

    zero16 = jnp.zeros((16,), jnp.float32)

    def _zero_den(r, _):
        for j in range(8):
            den_tab[r, pl.ds(j * 16, 16)] = zero16
        return 0
    lax.fori_loop(0, DEN_ROWS, _zero_den, 0)

    def _zero_rows(i, _):
        for j in range(8):
            g0[i, pl.ds(j * 16, 16)] = zero16
        return 0
    lax.fori_loop(0, CB, _zero_rows, 0)

    r0 = sid * (NP // NSUB)
    for c in range((NP // NSUB) // CB):
        pltpu.sync_copy(g0, sh_out.at[pl.ds(r0 + c * CB, CB)])

    @pl.when(sid == 0)
    def _():
        pltpu.sync_copy(den_tab, sh_den)

    ii = lax.iota(jnp.int32, 16)
    iota_v[0, pl.ds(0, 16)] = ii
    iota_v[0, pl.ds(16, 16)] = ii + 16
    iota_v[0, pl.ds(32, 16)] = ii + 32

    def _expass(b, _):
        for g in range(CB // 16):
            sl = pl.ds(b * CB + g * 16, 16)
            idx16 = src_all[sl]
            s16 = plsc.load_gather(s_tab, [idx16])
            x = s16 + ex_all[sl]
            e = jnp.maximum(x, x * 0.01)
            ex = jnp.exp(e)
            ex_all[sl] = ex
            d16 = dst_all[b, pl.ds(g * 16, 16)]
            plsc.addupdate_scatter(
                den_tab,
                [lax.shift_right_logical(d16, 7), lax.bitwise_and(d16, 127)],
                ex)
        return 0
    with jax.named_scope("sc_expass"):
        lax.fori_loop(0, NB, _expass, 0)

    plsc.subcore_barrier()

    gbufs = (g0, g1)
    sbufs = (sb0, sb1)
    gsems = (gsem0, gsem1)
    ssems = (ssem0, ssem1)

    def _start_gather(j, b):
        pltpu.async_copy(
            z_hbm.at[src_all.at[pl.ds(b * CB, CB)]], gbufs[j], gsems[j])

    def _scale(j, b):
        gb, sb = gbufs[j], sbufs[j]

        def body(g, _):
            ex16 = ex_all[pl.ds(b * CB + g * 16, 16)]
            for l in range(16):
                i = g * 16 + l
                vx = jnp.full((16,), ex16[l], jnp.float32)
                for jj in range(8):
                    sl2 = pl.ds(jj * 16, 16)
                    sb[i, sl2] = gb[i, sl2] * vx
            return 0
        lax.fori_loop(0, CB // 16, body, 0)

    _start_gather(0, 0)
    _start_gather(1, 1)

    def _pair(k, _):
        for j in range(2):
            b = 2 * k + j
            pltpu.make_async_copy(
                z_hbm.at[src_all.at[pl.ds(b * CB, CB)]],
                gbufs[j], gsems[j]).wait()
            @pl.when(k > 0)
            def _():
                pltpu.make_async_copy(
                    sbufs[j], sh_out.at[dst_all.at[b - 2]], ssems[j]).wait()
            _scale(j, b)
            pltpu.async_copy(
                sbufs[j], sh_out.at[dst_all.at[b]], ssems[j], add=True)
            @pl.when(b + 2 < NB)
            def _():
                pltpu.async_copy(
                    z_hbm.at[src_all.at[pl.ds((b + 2) * CB, CB)]],
                    gbufs[j], gsems[j])
        return 0
    with jax.named_scope("sc_rowpipe"):
        lax.fori_loop(0, NB // 2, _pair, 0)

        for j in range(2):
            pltpu.make_async_copy(
                sbufs[j], sh_out.at[dst_all.at[NB - 2 + j]], ssems[j]).wait()

    plsc.subcore_barrier()

    pltpu.sync_copy(den_tab, sh_den.at[iota_v.at[0]], add=True)

    plsc.subcore_barrier()

    pltpu.sync_copy(sh_out.at[pl.ds(r0, NP // NSUB)],
                    out_hbm.at[cid, pl.ds(r0, NP // NSUB)])

    @pl.when(sid == 0)
    def _():
        pltpu.sync_copy(sh_den, den_hbm.at[cid])


def _edge_call(src_p, dst_p, f_p, ssrc, z):
    mesh = plsc.VectorSubcoreMesh(core_axis_name="c", subcore_axis_name="s")
    fn = pl.kernel(
        _edge_body,
        out_type=[
            jax.ShapeDtypeStruct((NCORES, NP, OUT), jnp.float32),
            jax.ShapeDtypeStruct((NCORES, DEN_ROWS, 128), jnp.float32),
        ],
        mesh=mesh,
        scratch_types=[
            pltpu.VMEM((NP,), jnp.float32),
            pltpu.VMEM((DEN_ROWS, 128), jnp.float32),
            pltpu.VMEM((EW,), jnp.int32),
            pltpu.VMEM((NB, CB), jnp.int32),
            pltpu.VMEM((EW,), jnp.float32),
            pltpu.VMEM((CB, OUT), jnp.float32),
            pltpu.VMEM((CB, OUT), jnp.float32),
            pltpu.VMEM((CB, OUT), jnp.float32),
            pltpu.VMEM((CB, OUT), jnp.float32),
            pltpu.VMEM_SHARED((NP, OUT), jnp.float32),
            pltpu.VMEM_SHARED((DEN_ROWS, 128), jnp.float32),
            pltpu.VMEM((1, DEN_ROWS), jnp.int32),
            pltpu.SemaphoreType.DMA,
            pltpu.SemaphoreType.DMA,
            pltpu.SemaphoreType.DMA,
            pltpu.SemaphoreType.DMA,
        ],
        compiler_params=pltpu.CompilerParams(
            needs_layout_passes=False, use_tc_tiling_on_sc=True),
    )
    return fn(src_p, dst_p, f_p, ssrc, z)


def _norm_body(p_ref, d_ref, o_ref):
    p = p_ref[0] + p_ref[1]
    d = d_ref[0] + d_ref[1]
    d = jnp.where(d > 0.0, d, 1.0)
    o_ref[...] = p * (1.0 / d)[:, None]


def _norm_call(outp, den):
    grid = 20
    rb = NP // grid
    return pl.pallas_call(
        _norm_body,
        grid=(grid,),
        in_specs=[
            pl.BlockSpec((NCORES, rb, OUT), lambda i: (0, i, 0)),
            pl.BlockSpec((NCORES, rb), lambda i: (0, i)),
        ],
        out_specs=pl.BlockSpec((rb, OUT), lambda i: (i, 0)),
        out_shape=jax.ShapeDtypeStruct((NP, OUT), jnp.float32),
    )(outp, den)


def kernel(h, edge_index, tfidfembed, W_fc, W_feat, W_attn):
    src = edge_index[0]
    dst = edge_index[1]
    h_p = jnp.pad(h[:N_W], ((0, NP - N_W), (0, 0)))
    z, ssrc, f_p = _dense_call(h_p, W_fc.T, tfidfembed, W_attn, W_feat)
    pad_ids = jnp.arange(EP - E, dtype=jnp.int32)
    src_p = jnp.concatenate([src, pad_ids % N_W])
    dst_p = jnp.concatenate(
        [dst, N_S + pad_ids % (NP - N_S)]).reshape(EP // CB, CB)
    outp, denp = _edge_call(src_p, dst_p, f_p, ssrc, z)
    den = denp.reshape(NCORES, DEN_ROWS * 128)[:, :NP]
    res = _norm_call(outp, den)
    return res[:N_S]
